# Initial kernel scaffold; baseline (speedup 1.0000x reference)
import jax
import jax.numpy as jnp
from jax.experimental import pallas as pl


def kernel(x, edge_index, batch, params):
    raise NotImplementedError("write your pallas kernel here")


# speedup vs baseline: 1.4640x; 1.4640x over previous
import functools

import jax
import jax.numpy as jnp
from jax import lax
from jax.experimental import pallas as pl
from jax.experimental.pallas import tpu as pltpu
from jax.experimental.pallas import tpu_sc as plsc

N_NODES = 10000
N_EDGES = 320000
NUM_GRAPHS = 64
EPS = 1e-5
NEG_SLOPE = 0.01

_NC = 2
_NS = 16
_NW = _NC * _NS
_RPT = 312
_LAST_RPT = N_NODES - (_NW - 1) * _RPT
_SCROWS = _NS * _RPT
_TRASH = 5008
_ACCROWS = 5016
_ECHUNK = 512
_NCHUNKS = N_EDGES // _ECHUNK
_VPC = _ECHUNK // 16
_STAGE = 1024
_FLUSH_AT = _STAGE - 16
_LCAP = 327680
_GCHUNK = 64


def _sc_mesh():
  return plsc.VectorSubcoreMesh(core_axis_name="c", subcore_axis_name="s")


@functools.lru_cache(maxsize=None)
def _make_sc_scan():
  @functools.partial(
      pl.kernel,
      mesh=_sc_mesh(),
      compiler_params=pltpu.CompilerParams(needs_layout_passes=False),
      out_type=[
          jax.ShapeDtypeStruct((_NW * _LCAP,), jnp.int32),
          jax.ShapeDtypeStruct((_NW * _LCAP,), jnp.int32),
          jax.ShapeDtypeStruct((_NW * 16,), jnp.int32),
      ],
      scratch_types=[
          pltpu.VMEM((_ECHUNK,), jnp.int32),
          pltpu.VMEM((_ECHUNK,), jnp.int32),
          pltpu.VMEM((_STAGE,), jnp.int32),
          pltpu.VMEM((_STAGE,), jnp.int32),
          pltpu.VMEM((16,), jnp.int32),
          pltpu.SMEM((2,), jnp.int32),
      ],
  )
  def scan(src_hbm, dst_hbm, lsrc_hbm, ldst_hbm, cnt_hbm,
           srcv, dstv, st_s, st_d, cntv, state):
    cid = lax.axis_index("c")
    sid = lax.axis_index("s")
    wid = cid * _NS + sid
    lo = wid * _RPT
    hi = jnp.where(wid == _NW - 1, N_NODES, lo + _RPT)
    sc_base = cid * _SCROWS
    lbase = wid * _LCAP

    zeros16 = jnp.zeros((16,), jnp.int32)
    trash16 = jnp.full((16,), _TRASH, jnp.int32)

    def fill_stage():
      def body(k, c):
        kk = pl.multiple_of(k * 16, 16)
        st_s[pl.ds(kk, 16)] = zeros16
        st_d[pl.ds(kk, 16)] = trash16
        return c
      lax.fori_loop(0, _STAGE // 16, body, 0)

    def flush():
      o = pl.multiple_of(lbase + state[1], _STAGE)
      pltpu.sync_copy(st_s, lsrc_hbm.at[pl.ds(o, _STAGE)])
      pltpu.sync_copy(st_d, ldst_hbm.at[pl.ds(o, _STAGE)])
      fill_stage()
      state[1] = state[1] + _STAGE
      state[0] = 0

    fill_stage()
    state[0] = 0
    state[1] = 0

    def chunk_body(c, carry):
      cc = pl.multiple_of(c * _ECHUNK, _ECHUNK)
      pltpu.sync_copy(src_hbm.at[pl.ds(cc, _ECHUNK)], srcv)
      pltpu.sync_copy(dst_hbm.at[pl.ds(cc, _ECHUNK)], dstv)

      def vreg_body(i, carry2):
        @pl.when(state[0] >= _FLUSH_AT)
        def _():
          flush()

        n2 = state[0]
        ii = pl.multiple_of(i * 16, 16)
        s16 = srcv[pl.ds(ii, 16)]
        d16 = dstv[pl.ds(ii, 16)]
        lo_v = jnp.full((16,), lo, jnp.int32)
        hi_v = jnp.full((16,), hi, jnp.int32)
        m = jnp.logical_and(d16 >= lo_v, d16 < hi_v)
        one = jnp.full((16,), 1, jnp.int32)
        zero16i = jnp.full((16,), 0, jnp.int32)
        mi = jnp.where(m, one, zero16i)
        cnt = jnp.sum(mi)
        pos = jnp.full((16,), n2, jnp.int32) + plsc.cumsum(mi) - one
        scb = jnp.full((16,), sc_base, jnp.int32)
        plsc.store_scatter(st_s, [pos], s16, mask=m)
        plsc.store_scatter(st_d, [pos], d16 - scb, mask=m)
        state[0] = n2 + cnt
        return carry2

      return lax.fori_loop(0, _VPC, vreg_body, carry)

    lax.fori_loop(0, _NCHUNKS, chunk_body, 0)
    flush()
    cntv[...] = jnp.full((16,), state[1] // _GCHUNK, jnp.int32)
    pltpu.sync_copy(cntv, cnt_hbm.at[pl.ds(pl.multiple_of(wid * 16, 16), 16)])

  return scan


@functools.lru_cache(maxsize=None)
def _make_sc_agg():
  @functools.partial(
      pl.kernel,
      mesh=_sc_mesh(),
      compiler_params=pltpu.CompilerParams(needs_layout_passes=False),
      out_type=jax.ShapeDtypeStruct((N_NODES, 128), jnp.float32),
      scratch_types=[
          pltpu.VMEM((_GCHUNK,), jnp.int32),
          pltpu.VMEM((_GCHUNK,), jnp.int32),
          pltpu.VMEM((16,), jnp.int32),
          pltpu.VMEM((_GCHUNK, 128), jnp.float32),
          pltpu.VMEM_SHARED((_ACCROWS, 128), jnp.float32),
          pltpu.SemaphoreType.DMA,
      ],
  )
  def agg(h_hbm, lsrc_hbm, ldst_hbm, cnt_hbm, zero_hbm, out_hbm,
          gidx, sidx, cntv, rows, acc, sem):
    cid = lax.axis_index("c")
    sid = lax.axis_index("s")
    wid = cid * _NS + sid
    lbase = wid * _LCAP
    rbase = sid * _RPT

    @pl.when(wid < _NW - 1)
    def _():
      pltpu.sync_copy(zero_hbm.at[pl.ds(0, _RPT)], acc.at[pl.ds(rbase, _RPT)])

    @pl.when(wid == _NW - 1)
    def _():
      pltpu.sync_copy(zero_hbm.at[pl.ds(0, _LAST_RPT)],
                      acc.at[pl.ds(rbase, _LAST_RPT)])

    @pl.when(sid == 0)
    def _():
      pltpu.sync_copy(zero_hbm.at[pl.ds(0, 8)], acc.at[pl.ds(_TRASH, 8)])

    plsc.subcore_barrier()

    pltpu.sync_copy(cnt_hbm.at[pl.ds(pl.multiple_of(wid * 16, 16), 16)], cntv)
    nchunks = cntv[...][0]

    def body(c, carry):
      off = pl.multiple_of(lbase + c * _GCHUNK, _GCHUNK)
      pltpu.sync_copy(lsrc_hbm.at[pl.ds(off, _GCHUNK)], gidx)
      pltpu.sync_copy(ldst_hbm.at[pl.ds(off, _GCHUNK)], sidx)
      pltpu.async_copy(h_hbm.at[gidx], rows, sem).wait()
      pltpu.sync_copy(rows, acc.at[sidx], add=True)
      return carry

    lax.fori_loop(0, nchunks, body, 0)

    plsc.subcore_barrier()

    gbase = wid * _RPT

    @pl.when(wid < _NW - 1)
    def _():
      pltpu.sync_copy(acc.at[pl.ds(rbase, _RPT)],
                      out_hbm.at[pl.ds(gbase, _RPT)])

    @pl.when(wid == _NW - 1)
    def _():
      pltpu.sync_copy(acc.at[pl.ds(rbase, _LAST_RPT)],
                      out_hbm.at[pl.ds(gbase, _LAST_RPT)])

  return agg


def _leaky(x):
  return jnp.where(x >= 0, x, NEG_SLOPE * x)


def _matmul_t(a, w):
  return lax.dot_general(a, w, (((1,), (1,)), ((), ())))


def _tc_pre_body(c_in, agg_ref, h_ref, wrel_ref, brel_ref, wroot_ref,
                 wlin_ref, blin_ref, out_ref):
  agg = agg_ref[:, :c_in]
  h = h_ref[:, :c_in]
  t = _matmul_t(agg, wrel_ref[...]) + brel_ref[...] + _matmul_t(h, wroot_ref[...])
  t = _leaky(t)
  out_ref[...] = _matmul_t(t, wlin_ref[...]) + blin_ref[...]


def _tc_pre(agg, h, lp):
  c_in = lp['W_rel'].shape[1]
  c_out = lp['W_rel'].shape[0]
  return pl.pallas_call(
      functools.partial(_tc_pre_body, c_in),
      out_shape=jax.ShapeDtypeStruct((N_NODES, c_out), jnp.float32),
      compiler_params=pltpu.CompilerParams(vmem_limit_bytes=120 * 1024 * 1024),
  )(agg, h, lp['W_rel'], lp['b_rel'].reshape(1, -1), lp['W_root'],
    lp['W_lin'], lp['b_lin'].reshape(1, -1))


def _tc_post_body(pad_out, u_ref, m_ref, v_ref, g_ref, b_ref, out_ref):
  hh = ((u_ref[...] - m_ref[...]) * lax.rsqrt(v_ref[...] + EPS)
        * g_ref[...] + b_ref[...])
  res = _leaky(hh)
  if pad_out:
    res = jnp.concatenate(
        [res, jnp.zeros((N_NODES, pad_out), jnp.float32)], axis=1)
  out_ref[...] = res


def _tc_post(u, m, v, lp):
  c_out = u.shape[1]
  out_w = max(c_out, 128)
  return pl.pallas_call(
      functools.partial(_tc_post_body, out_w - c_out),
      out_shape=jax.ShapeDtypeStruct((N_NODES, out_w), jnp.float32),
      compiler_params=pltpu.CompilerParams(vmem_limit_bytes=120 * 1024 * 1024),
  )(u, m.reshape(1, -1), v.reshape(1, -1), lp['bn_g'].reshape(1, -1),
    lp['bn_b'].reshape(1, -1))


def _tc_pool_body(h_ref, batch_ref, w_ref, b_ref, out_ref):
  h = h_ref[...]
  b = batch_ref[...]
  gids = lax.broadcasted_iota(jnp.int32, (NUM_GRAPHS, N_NODES), 0)
  oh = (gids == b).astype(jnp.float32)
  sums = jnp.dot(oh, h, precision=lax.Precision.HIGHEST)
  cnt = jnp.sum(oh, axis=1, keepdims=True)
  z = sums / jnp.maximum(cnt, 1.0)
  out_ref[...] = _matmul_t(z, w_ref[...]) + b_ref[...]


def _tc_pool(h, batch32, lin0):
  return pl.pallas_call(
      _tc_pool_body,
      out_shape=jax.ShapeDtypeStruct((NUM_GRAPHS, lin0['W'].shape[0]),
                                     jnp.float32),
      compiler_params=pltpu.CompilerParams(vmem_limit_bytes=120 * 1024 * 1024),
  )(h, batch32, lin0['W'], lin0['b'].reshape(1, -1))


def _tc_mlp_body(z_ref, m_ref, v_ref, g_ref, b_ref, w_ref, bb_ref, out_ref):
  zn = ((z_ref[...] - m_ref[...]) * lax.rsqrt(v_ref[...] + EPS)
        * g_ref[...] + b_ref[...])
  zn = jnp.maximum(zn, 0.0)
  out_ref[...] = _matmul_t(zn, w_ref[...]) + bb_ref[...]


def _tc_mlp(z, m, v, bn, lin):
  return pl.pallas_call(
      _tc_mlp_body,
      out_shape=jax.ShapeDtypeStruct((NUM_GRAPHS, lin['W'].shape[0]),
                                     jnp.float32),
  )(z, m.reshape(1, -1), v.reshape(1, -1), bn['g'].reshape(1, -1),
    bn['b'].reshape(1, -1), lin['W'], lin['b'].reshape(1, -1))


def kernel(x, edge_index, batch, params):
  src = edge_index[0].astype(jnp.int32)
  dst = edge_index[1].astype(jnp.int32)
  batch32 = batch.astype(jnp.int32).reshape(1, N_NODES)
  zero = jnp.zeros((_LAST_RPT, 128), jnp.float32)

  lsrc, ldst, cnts = _make_sc_scan()(src, dst)

  h = x
  for lp in params['layers']:
    agg = _make_sc_agg()(h, lsrc, ldst, cnts, zero)
    u = _tc_pre(agg, h, lp)
    m = jnp.mean(u, axis=0)
    v = jnp.var(u, axis=0)
    h = _tc_post(u, m, v, lp)

  mlp_lins = params['mlp_lins']
  mlp_bns = params['mlp_bns']
  z = _tc_pool(h, batch32, mlp_lins[0])
  for i in range(1, len(mlp_lins)):
    m = jnp.mean(z, axis=0)
    v = jnp.var(z, axis=0)
    z = _tc_mlp(z, m, v, mlp_bns[i - 1], mlp_lins[i])
  return z
